# baseline (speedup 1.0000x reference)
import functools

import jax
import jax.numpy as jnp
from jax import lax
from jax.experimental import pallas as pl
from jax.experimental.pallas import tpu as pltpu
from jax.experimental.pallas import tpu_sc as plsc

NU = 25000
NI = 25000
NN = NU + NI
D = 64
HW = 32
E = 800000

SUB = 128
JPC = 12
CH = SUB * JPC
NT = 16
NCH = -(-E // (NT * CH))
EPT = NCH * CH
EPAD = EPT * NT
STRIPE = 1568
ACC_R = STRIPE * NT
DUMMY = NI
CPO = 1560
CPO_TAIL = NN // 2 - NT * CPO

_MESH = plsc.VectorSubcoreMesh(core_axis_name="c", subcore_axis_name="s")
_SC_PARAMS = pltpu.CompilerParams(use_tc_tiling_on_sc=False)


def _zero_fill(zb, rows, cols):
    def body(i, _):
        for j in range(cols // 16):
            zb[i, pl.ds(j * 16, 16)] = jnp.zeros((16,), jnp.float32)
        return 0
    lax.fori_loop(0, rows, body, 0)


def _spmv6(gidx, sidx, ys):
    @functools.partial(
        pl.kernel,
        mesh=_MESH,
        compiler_params=_SC_PARAMS,
        out_type=[jax.ShapeDtypeStruct((NN, HW), jnp.float32)] * 6,
        scratch_types=[
            pltpu.VMEM((JPC, SUB), jnp.int32),
            pltpu.VMEM((JPC, SUB), jnp.int32),
            pltpu.VMEM((JPC, SUB, HW), jnp.float32),
            pltpu.VMEM((256, HW), jnp.float32),
            pltpu.VMEM((256, HW), jnp.float32),
            pltpu.VMEM_SHARED((ACC_R, HW), jnp.float32),
            pltpu.SemaphoreType.DMA,
            pltpu.SemaphoreType.DMA,
            pltpu.SemaphoreType.DMA,
        ],
    )
    def k(gidx_h, sidx_h, y0, y1, y2, y3, y4, y5, o0, o1, o2, o3, o4, o5,
          gi, si, rows, zb, stg, acc, isem, gsem, ssem):
        core = lax.axis_index("c")
        sid = lax.axis_index("s")
        _zero_fill(zb, 256, HW)
        out_base = (1 - core) * NI

        for y_h, o_h in ((y0, o0), (y1, o1), (y2, o2),
                         (y3, o3), (y4, o4), (y5, o5)):
            for q in range(6):
                pltpu.sync_copy(zb, acc.at[pl.ds(sid * STRIPE + q * 256, 256)])
            pltpu.sync_copy(zb.at[pl.ds(0, 32)],
                            acc.at[pl.ds(sid * STRIPE + 1536, 32)])
            plsc.subcore_barrier()

            def chunk(c, _):
                base = sid * (NCH * JPC) + c * JPC
                i1 = pltpu.async_copy(gidx_h.at[core, pl.ds(base, JPC)],
                                      gi, isem)
                i2 = pltpu.async_copy(sidx_h.at[core, pl.ds(base, JPC)],
                                      si, isem)
                i1.wait()
                i2.wait()
                gds = [pltpu.async_copy(y_h.at[gi.at[j]], rows.at[j], gsem)
                       for j in range(JPC)]
                sds = []
                for j in range(JPC):
                    gds[j].wait()
                    sds.append(pltpu.async_copy(rows.at[j], acc.at[si.at[j]],
                                                ssem, add=True))
                for d in sds:
                    d.wait()
                return 0

            lax.fori_loop(0, NCH, chunk, 0)
            plsc.subcore_barrier()

            for q in range(6):
                pltpu.sync_copy(acc.at[pl.ds(sid * CPO + q * 256, 256)], stg)
                pltpu.sync_copy(
                    stg, o_h.at[pl.ds(out_base + sid * CPO + q * 256, 256)])
            pltpu.sync_copy(acc.at[pl.ds(sid * CPO + 1536, CPO - 1536)],
                            stg.at[pl.ds(0, CPO - 1536)])
            pltpu.sync_copy(
                stg.at[pl.ds(0, CPO - 1536)],
                o_h.at[pl.ds(out_base + sid * CPO + 1536, CPO - 1536)])

            @pl.when(sid == 0)
            def _():
                pltpu.sync_copy(acc.at[pl.ds(NT * CPO, CPO_TAIL)],
                                stg.at[pl.ds(0, CPO_TAIL)])
                pltpu.sync_copy(
                    stg.at[pl.ds(0, CPO_TAIL)],
                    o_h.at[pl.ds(out_base + NT * CPO, CPO_TAIL)])

            plsc.subcore_barrier()

    return k(gidx, sidx, *ys)


def _degrees(sidx):
    @functools.partial(
        pl.kernel,
        mesh=_MESH,
        compiler_params=_SC_PARAMS,
        out_type=jax.ShapeDtypeStruct((2 * ACC_R,), jnp.float32),
        scratch_types=[
            pltpu.VMEM((JPC, SUB), jnp.int32),
            pltpu.VMEM((SUB,), jnp.float32),
            pltpu.VMEM((STRIPE,), jnp.float32),
            pltpu.VMEM_SHARED((ACC_R,), jnp.float32),
        ],
    )
    def k(sidx_h, o_h, si, ones, z1, dacc):
        core = lax.axis_index("c")
        sid = lax.axis_index("s")

        def fill_ones(i, _):
            ones[pl.ds(i * 16, 16)] = jnp.ones((16,), jnp.float32)
            return 0
        lax.fori_loop(0, SUB // 16, fill_ones, 0)

        def fill_z(i, _):
            z1[pl.ds(i * 16, 16)] = jnp.zeros((16,), jnp.float32)
            return 0
        lax.fori_loop(0, STRIPE // 16, fill_z, 0)

        pltpu.sync_copy(z1, dacc.at[pl.ds(sid * STRIPE, STRIPE)])
        plsc.subcore_barrier()

        def chunk(c, _):
            base = sid * (NCH * JPC) + c * JPC
            pltpu.sync_copy(sidx_h.at[core, pl.ds(base, JPC)], si)
            for j in range(JPC):
                pltpu.sync_copy(ones, dacc.at[si.at[j]], add=True)
            return 0

        lax.fori_loop(0, NCH, chunk, 0)
        plsc.subcore_barrier()
        pltpu.sync_copy(dacc.at[pl.ds(sid * STRIPE, STRIPE)], z1)
        pltpu.sync_copy(z1,
                        o_h.at[pl.ds(core * ACC_R + sid * STRIPE, STRIPE)])

    return k(sidx)


_BLK = 1000


def _feat_proj(v_feat, t_feat, W_img, b_img, W_txt, b_txt):
    def body(vf, tf, wi, bi, wt, bt, vis, txt):
        vis[...] = jnp.dot(vf[...], wi[...],
                           preferred_element_type=jnp.float32) + bi[...]
        txt[...] = jnp.dot(tf[...], wt[...],
                           preferred_element_type=jnp.float32) + bt[...]

    return pl.pallas_call(
        body,
        grid=(NI // _BLK,),
        in_specs=[
            pl.BlockSpec((_BLK, 512), lambda i: (i, 0)),
            pl.BlockSpec((_BLK, 384), lambda i: (i, 0)),
            pl.BlockSpec((512, D), lambda i: (0, 0)),
            pl.BlockSpec((1, D), lambda i: (0, 0)),
            pl.BlockSpec((384, D), lambda i: (0, 0)),
            pl.BlockSpec((1, D), lambda i: (0, 0)),
        ],
        out_specs=[pl.BlockSpec((_BLK, D), lambda i: (i, 0))] * 2,
        out_shape=[jax.ShapeDtypeStruct((NI, D), jnp.float32)] * 2,
    )(v_feat, t_feat, W_img, b_img.reshape(1, D), W_txt, b_txt.reshape(1, D))


_X_SPEC = pl.BlockSpec((_BLK, D), lambda i: (i, 0))
_H_SPEC = pl.BlockSpec((_BLK, HW), lambda i: (i, 0))
_W_SPEC = pl.BlockSpec((D, D), lambda i: (0, 0))
_D_SPEC = pl.BlockSpec((_BLK, 1), lambda i: (i, 0))
_Y_SHAPES = [jax.ShapeDtypeStruct((NN, HW), jnp.float32)] * 6


def _y0(x_g, x_v, x_t, Wg, Wm, degc):
    def body(xg, xv, xt, dg, wg, wm, yg0, yg1, yv0, yv1, yt0, yt1):
        dinv = lax.rsqrt(dg[...] + 1.0)
        yg = dinv * jnp.dot(xg[...], wg[...],
                            preferred_element_type=jnp.float32)
        yv = dinv * jnp.dot(xv[...], wm[...],
                            preferred_element_type=jnp.float32)
        yt = dinv * jnp.dot(xt[...], wm[...],
                            preferred_element_type=jnp.float32)
        yg0[...] = yg[:, :HW]
        yg1[...] = yg[:, HW:]
        yv0[...] = yv[:, :HW]
        yv1[...] = yv[:, HW:]
        yt0[...] = yt[:, :HW]
        yt1[...] = yt[:, HW:]

    return pl.pallas_call(
        body,
        grid=(NN // _BLK,),
        in_specs=[_X_SPEC, _X_SPEC, _X_SPEC, _D_SPEC, _W_SPEC, _W_SPEC],
        out_specs=[_H_SPEC] * 6,
        out_shape=_Y_SHAPES,
    )(x_g, x_v, x_t, degc, Wg, Wm)


def _mid(accs, ys, degc, Wg, Wm, s_g, s_v, s_t):
    def body(a0, a1, a2, a3, a4, a5, y0, y1, y2, y3, y4, y5,
             dg, wg, wm, sg, sv, st,
             yg0, yg1, yv0, yv1, yt0, yt1, sg2, sv2, st2):
        dinv = lax.rsqrt(dg[...] + 1.0)
        hg = jnp.concatenate(
            [dinv * (a0[...] + y0[...]), dinv * (a1[...] + y1[...])], axis=1)
        hv = jnp.concatenate(
            [dinv * (a2[...] + y2[...]), dinv * (a3[...] + y3[...])], axis=1)
        ht = jnp.concatenate(
            [dinv * (a4[...] + y4[...]), dinv * (a5[...] + y5[...])], axis=1)
        sg2[...] = sg[...] + hg
        sv2[...] = sv[...] + hv
        st2[...] = st[...] + ht
        yg = dinv * jnp.dot(hg, wg[...], preferred_element_type=jnp.float32)
        yv = dinv * jnp.dot(hv, wm[...], preferred_element_type=jnp.float32)
        yt = dinv * jnp.dot(ht, wm[...], preferred_element_type=jnp.float32)
        yg0[...] = yg[:, :HW]
        yg1[...] = yg[:, HW:]
        yv0[...] = yv[:, :HW]
        yv1[...] = yv[:, HW:]
        yt0[...] = yt[:, :HW]
        yt1[...] = yt[:, HW:]

    return pl.pallas_call(
        body,
        grid=(NN // _BLK,),
        in_specs=[_H_SPEC] * 12 + [_D_SPEC, _W_SPEC, _W_SPEC] + [_X_SPEC] * 3,
        out_specs=[_H_SPEC] * 6 + [_X_SPEC] * 3,
        out_shape=_Y_SHAPES + [jax.ShapeDtypeStruct((NN, D), jnp.float32)] * 3,
    )(*accs, *ys, degc, Wg, Wm, s_g, s_v, s_t)


def _fin(accs, ys, degc, s_g, s_v, s_t):
    def body(a0, a1, a2, a3, a4, a5, y0, y1, y2, y3, y4, y5,
             dg, sg, sv, st, og, ov, ot):
        dinv = lax.rsqrt(dg[...] + 1.0)
        hg = jnp.concatenate(
            [dinv * (a0[...] + y0[...]), dinv * (a1[...] + y1[...])], axis=1)
        hv = jnp.concatenate(
            [dinv * (a2[...] + y2[...]), dinv * (a3[...] + y3[...])], axis=1)
        ht = jnp.concatenate(
            [dinv * (a4[...] + y4[...]), dinv * (a5[...] + y5[...])], axis=1)
        og[...] = (sg[...] + hg) * 0.25
        ov[...] = (sv[...] + hv) * 0.25
        ot[...] = (st[...] + ht) * 0.25

    return pl.pallas_call(
        body,
        grid=(NN // _BLK,),
        in_specs=[_H_SPEC] * 12 + [_D_SPEC] + [_X_SPEC] * 3,
        out_specs=[_X_SPEC] * 3,
        out_shape=[jax.ShapeDtypeStruct((NN, D), jnp.float32)] * 3,
    )(*accs, *ys, degc, s_g, s_v, s_t)


def kernel(user_emb, item_emb, v_feat, t_feat, W_img, b_img, W_txt, b_txt,
           Wg0, Wg1, Wg2, Wm0, Wm1, Wm2, edge_user, edge_item):
    eu = edge_user.astype(jnp.int32)
    ei = edge_item.astype(jnp.int32)
    pad = EPAD - E
    zpad = jnp.zeros((pad,), jnp.int32)
    dpad = jnp.full((pad,), DUMMY, jnp.int32)
    gidx = jnp.stack([jnp.concatenate([eu, zpad]),
                      jnp.concatenate([ei + NU, zpad + NU])]
                     ).reshape(2, EPAD // SUB, SUB)
    sidx = jnp.stack([jnp.concatenate([ei, dpad]),
                      jnp.concatenate([eu, dpad])]
                     ).reshape(2, EPAD // SUB, SUB)

    cnt = _degrees(sidx)
    degc = jnp.concatenate([cnt[ACC_R:ACC_R + NU],
                            cnt[:NI]]).reshape(NN, 1)

    vis, txt = _feat_proj(v_feat, t_feat, W_img, b_img, W_txt, b_txt)
    ego_g = jnp.concatenate([user_emb, item_emb], axis=0)
    ego_v = jnp.concatenate([user_emb, vis], axis=0)
    ego_t = jnp.concatenate([user_emb, txt], axis=0)

    ys = _y0(ego_g, ego_v, ego_t, Wg0, Wm0, degc)
    accs = _spmv6(gidx, sidx, ys)
    *ys, s_g, s_v, s_t = _mid(accs, ys, degc, Wg1, Wm1, ego_g, ego_v, ego_t)
    accs = _spmv6(gidx, sidx, ys)
    *ys, s_g, s_v, s_t = _mid(accs, ys, degc, Wg2, Wm2, s_g, s_v, s_t)
    accs = _spmv6(gidx, sidx, ys)
    o_g, o_v, o_t = _fin(accs, ys, degc, s_g, s_v, s_t)

    return jnp.concatenate([o_g, o_v, o_t], axis=0)


# speedup vs baseline: 1.5010x; 1.5010x over previous
import functools

import jax
import jax.numpy as jnp
from jax import lax
from jax.experimental import pallas as pl
from jax.experimental.pallas import tpu as pltpu
from jax.experimental.pallas import tpu_sc as plsc

NU = 25000
NI = 25000
NN = NU + NI
D = 64
HW = 32
E = 800000

SUB = 128
JPC = 8
CH = SUB * JPC
NT = 16
NCH = -(-E // (NT * CH))
EPT = NCH * CH
EPAD = EPT * NT
STRIPE = 1568
ACC_R = STRIPE * NT
DUMMY = NI
CPO = 1560
CPO_TAIL = NN // 2 - NT * CPO

_MESH = plsc.VectorSubcoreMesh(core_axis_name="c", subcore_axis_name="s")
_SC_PARAMS = pltpu.CompilerParams(use_tc_tiling_on_sc=False)


def _zero_fill(zb, rows, cols):
    def body(i, _):
        for j in range(cols // 16):
            zb[i, pl.ds(j * 16, 16)] = jnp.zeros((16,), jnp.float32)
        return 0
    lax.fori_loop(0, rows, body, 0)


def _spmv2(gidx, sidx, y_lo, y_hi):
    @functools.partial(
        pl.kernel,
        mesh=_MESH,
        compiler_params=_SC_PARAMS,
        out_type=[jax.ShapeDtypeStruct((NN, HW), jnp.float32)] * 2,
        scratch_types=[
            pltpu.VMEM((JPC, SUB), jnp.int32),
            pltpu.VMEM((JPC, SUB), jnp.int32),
            pltpu.VMEM((JPC, SUB, HW), jnp.float32),
            pltpu.VMEM((256, HW), jnp.float32),
            pltpu.VMEM((256, HW), jnp.float32),
            pltpu.VMEM_SHARED((ACC_R, HW), jnp.float32),
            pltpu.SemaphoreType.DMA,
            pltpu.SemaphoreType.DMA,
            pltpu.SemaphoreType.DMA,
        ],
    )
    def k(gidx_h, sidx_h, y0, y1, o0, o1,
          gi, si, rows, zb, stg, acc, isem, gsem, ssem):
        core = lax.axis_index("c")
        sid = lax.axis_index("s")
        _zero_fill(zb, 256, HW)
        out_base = (1 - core) * NI

        for y_h, o_h in ((y0, o0), (y1, o1)):
            for q in range(6):
                pltpu.sync_copy(zb, acc.at[pl.ds(sid * STRIPE + q * 256, 256)])
            pltpu.sync_copy(zb.at[pl.ds(0, 32)],
                            acc.at[pl.ds(sid * STRIPE + 1536, 32)])
            plsc.subcore_barrier()

            def chunk(c, _):
                base = sid * (NCH * JPC) + c * JPC
                i1 = pltpu.async_copy(gidx_h.at[core, pl.ds(base, JPC)],
                                      gi, isem)
                i2 = pltpu.async_copy(sidx_h.at[core, pl.ds(base, JPC)],
                                      si, isem)
                i1.wait()
                i2.wait()
                gds = [pltpu.async_copy(y_h.at[gi.at[j]], rows.at[j], gsem)
                       for j in range(JPC)]
                sds = []
                for j in range(JPC):
                    gds[j].wait()
                    sds.append(pltpu.async_copy(rows.at[j], acc.at[si.at[j]],
                                                ssem, add=True))
                for d in sds:
                    d.wait()
                return 0

            lax.fori_loop(0, NCH, chunk, 0)
            plsc.subcore_barrier()

            for q in range(6):
                pltpu.sync_copy(acc.at[pl.ds(sid * CPO + q * 256, 256)], stg)
                pltpu.sync_copy(
                    stg, o_h.at[pl.ds(out_base + sid * CPO + q * 256, 256)])
            pltpu.sync_copy(acc.at[pl.ds(sid * CPO + 1536, CPO - 1536)],
                            stg.at[pl.ds(0, CPO - 1536)])
            pltpu.sync_copy(
                stg.at[pl.ds(0, CPO - 1536)],
                o_h.at[pl.ds(out_base + sid * CPO + 1536, CPO - 1536)])

            @pl.when(sid == 0)
            def _():
                pltpu.sync_copy(acc.at[pl.ds(NT * CPO, CPO_TAIL)],
                                stg.at[pl.ds(0, CPO_TAIL)])
                pltpu.sync_copy(
                    stg.at[pl.ds(0, CPO_TAIL)],
                    o_h.at[pl.ds(out_base + NT * CPO, CPO_TAIL)])

            plsc.subcore_barrier()

    return k(gidx, sidx, y_lo, y_hi)


def _degrees(sidx):
    @functools.partial(
        pl.kernel,
        mesh=_MESH,
        compiler_params=_SC_PARAMS,
        out_type=jax.ShapeDtypeStruct((2 * ACC_R,), jnp.float32),
        scratch_types=[
            pltpu.VMEM((JPC, SUB), jnp.int32),
            pltpu.VMEM((SUB,), jnp.float32),
            pltpu.VMEM((STRIPE,), jnp.float32),
            pltpu.VMEM_SHARED((ACC_R,), jnp.float32),
        ],
    )
    def k(sidx_h, o_h, si, ones, z1, dacc):
        core = lax.axis_index("c")
        sid = lax.axis_index("s")

        def fill_ones(i, _):
            ones[pl.ds(i * 16, 16)] = jnp.ones((16,), jnp.float32)
            return 0
        lax.fori_loop(0, SUB // 16, fill_ones, 0)

        def fill_z(i, _):
            z1[pl.ds(i * 16, 16)] = jnp.zeros((16,), jnp.float32)
            return 0
        lax.fori_loop(0, STRIPE // 16, fill_z, 0)

        pltpu.sync_copy(z1, dacc.at[pl.ds(sid * STRIPE, STRIPE)])
        plsc.subcore_barrier()

        def chunk(c, _):
            base = sid * (NCH * JPC) + c * JPC
            pltpu.sync_copy(sidx_h.at[core, pl.ds(base, JPC)], si)
            for j in range(JPC):
                pltpu.sync_copy(ones, dacc.at[si.at[j]], add=True)
            return 0

        lax.fori_loop(0, NCH, chunk, 0)
        plsc.subcore_barrier()
        pltpu.sync_copy(dacc.at[pl.ds(sid * STRIPE, STRIPE)], z1)
        pltpu.sync_copy(z1,
                        o_h.at[pl.ds(core * ACC_R + sid * STRIPE, STRIPE)])

    return k(sidx)


_BLK = 1000


def _feat_proj(v_feat, t_feat, W_img, b_img, W_txt, b_txt):
    def body(vf, tf, wi, bi, wt, bt, vis, txt):
        vis[...] = jnp.dot(vf[...], wi[...],
                           preferred_element_type=jnp.float32) + bi[...]
        txt[...] = jnp.dot(tf[...], wt[...],
                           preferred_element_type=jnp.float32) + bt[...]

    return pl.pallas_call(
        body,
        grid=(NI // _BLK,),
        in_specs=[
            pl.BlockSpec((_BLK, 512), lambda i: (i, 0)),
            pl.BlockSpec((_BLK, 384), lambda i: (i, 0)),
            pl.BlockSpec((512, D), lambda i: (0, 0)),
            pl.BlockSpec((1, D), lambda i: (0, 0)),
            pl.BlockSpec((384, D), lambda i: (0, 0)),
            pl.BlockSpec((1, D), lambda i: (0, 0)),
        ],
        out_specs=[pl.BlockSpec((_BLK, D), lambda i: (i, 0))] * 2,
        out_shape=[jax.ShapeDtypeStruct((NI, D), jnp.float32)] * 2,
    )(v_feat, t_feat, W_img, b_img.reshape(1, D), W_txt, b_txt.reshape(1, D))


_X_SPEC = pl.BlockSpec((_BLK, D), lambda i: (i, 0))
_H_SPEC = pl.BlockSpec((_BLK, HW), lambda i: (i, 0))
_W_SPEC = pl.BlockSpec((D, D), lambda i: (0, 0))
_D_SPEC = pl.BlockSpec((_BLK, 1), lambda i: (i, 0))
_Y_SHAPES = [jax.ShapeDtypeStruct((NN, HW), jnp.float32)] * 2


def _y0(x, W, degc):
    def body(xr, dg, wr, ylo, yhi):
        dinv = lax.rsqrt(dg[...] + 1.0)
        y = dinv * jnp.dot(xr[...], wr[...],
                           preferred_element_type=jnp.float32)
        ylo[...] = y[:, :HW]
        yhi[...] = y[:, HW:]

    return pl.pallas_call(
        body,
        grid=(NN // _BLK,),
        in_specs=[_X_SPEC, _D_SPEC, _W_SPEC],
        out_specs=[_H_SPEC] * 2,
        out_shape=_Y_SHAPES,
    )(x, degc, W)


def _mid(a0, a1, y0, y1, degc, W, s_in):
    def body(ar0, ar1, yr0, yr1, dg, wr, sr, ylo, yhi, s2):
        dinv = lax.rsqrt(dg[...] + 1.0)
        h = jnp.concatenate(
            [dinv * (ar0[...] + yr0[...]), dinv * (ar1[...] + yr1[...])],
            axis=1)
        s2[...] = sr[...] + h
        y = dinv * jnp.dot(h, wr[...], preferred_element_type=jnp.float32)
        ylo[...] = y[:, :HW]
        yhi[...] = y[:, HW:]

    return pl.pallas_call(
        body,
        grid=(NN // _BLK,),
        in_specs=[_H_SPEC] * 4 + [_D_SPEC, _W_SPEC, _X_SPEC],
        out_specs=[_H_SPEC] * 2 + [_X_SPEC],
        out_shape=_Y_SHAPES + [jax.ShapeDtypeStruct((NN, D), jnp.float32)],
    )(a0, a1, y0, y1, degc, W, s_in)


def _fin(a0, a1, y0, y1, degc, s_in):
    def body(ar0, ar1, yr0, yr1, dg, sr, o):
        dinv = lax.rsqrt(dg[...] + 1.0)
        h = jnp.concatenate(
            [dinv * (ar0[...] + yr0[...]), dinv * (ar1[...] + yr1[...])],
            axis=1)
        o[...] = (sr[...] + h) * 0.25

    return pl.pallas_call(
        body,
        grid=(NN // _BLK,),
        in_specs=[_H_SPEC] * 4 + [_D_SPEC, _X_SPEC],
        out_specs=_X_SPEC,
        out_shape=jax.ShapeDtypeStruct((NN, D), jnp.float32),
    )(a0, a1, y0, y1, degc, s_in)


def kernel(user_emb, item_emb, v_feat, t_feat, W_img, b_img, W_txt, b_txt,
           Wg0, Wg1, Wg2, Wm0, Wm1, Wm2, edge_user, edge_item):
    eu = edge_user.astype(jnp.int32)
    ei = edge_item.astype(jnp.int32)
    pad = EPAD - E
    zpad = jnp.zeros((pad,), jnp.int32)
    dpad = jnp.full((pad,), DUMMY, jnp.int32)
    gidx = jnp.stack([jnp.concatenate([eu, zpad]),
                      jnp.concatenate([ei + NU, zpad + NU])]
                     ).reshape(2, EPAD // SUB, SUB)
    sidx = jnp.stack([jnp.concatenate([ei, dpad]),
                      jnp.concatenate([eu, dpad])]
                     ).reshape(2, EPAD // SUB, SUB)

    cnt = _degrees(sidx)
    degc = jnp.concatenate([cnt[ACC_R:ACC_R + NU],
                            cnt[:NI]]).reshape(NN, 1)

    vis, txt = _feat_proj(v_feat, t_feat, W_img, b_img, W_txt, b_txt)
    ego_g = jnp.concatenate([user_emb, item_emb], axis=0)
    ego_v = jnp.concatenate([user_emb, vis], axis=0)
    ego_t = jnp.concatenate([user_emb, txt], axis=0)

    outs = []
    for ego, Ws in ((ego_g, (Wg0, Wg1, Wg2)),
                    (ego_v, (Wm0, Wm1, Wm2)),
                    (ego_t, (Wm0, Wm1, Wm2))):
        y0, y1 = _y0(ego, Ws[0], degc)
        a0, a1 = _spmv2(gidx, sidx, y0, y1)
        y0, y1, s = _mid(a0, a1, y0, y1, degc, Ws[1], ego)
        a0, a1 = _spmv2(gidx, sidx, y0, y1)
        y0, y1, s = _mid(a0, a1, y0, y1, degc, Ws[2], s)
        a0, a1 = _spmv2(gidx, sidx, y0, y1)
        outs.append(_fin(a0, a1, y0, y1, degc, s))

    return jnp.concatenate(outs, axis=0)
